# Initial kernel scaffold; baseline (speedup 1.0000x reference)
import jax
import jax.numpy as jnp
from jax.experimental import pallas as pl


def kernel(inputs, embeddings):
    raise NotImplementedError("write your pallas kernel here")


# speedup vs baseline: 1.0717x; 1.0717x over previous
import functools

import jax
import jax.numpy as jnp
from jax import lax
from jax.experimental import pallas as pl
from jax.experimental.pallas import tpu as pltpu
from jax.experimental.pallas import tpu_sc as plsc

NUM_EMB = 8192
DIM = 64
ROWS = 16384
ROW_TILE = 256
N_ROW_TILES = ROWS // ROW_TILE
CHUNK = 2048
N_CHUNKS = NUM_EMB // CHUNK
_DOT_PRECISION = lax.Precision.DEFAULT


def _tc_body(x_ref, emb_ref, idx_ref, loss_ref):
    i = pl.program_id(0)
    x = x_ref[...]
    normsq = jnp.sum(x * x, axis=1, keepdims=True)
    norm = jnp.sqrt(normsq)
    xn = x / jnp.maximum(norm, 1e-12)

    def body(c, carry):
        best_val, best_idx = carry
        e = emb_ref[pl.ds(c * CHUNK, CHUNK), :]
        d = lax.dot_general(
            xn, e, (((1,), (1,)), ((), ())),
            preferred_element_type=jnp.float32,
            precision=_DOT_PRECISION,
        )
        m = jnp.max(d, axis=1)
        a = jnp.argmax(d, axis=1).astype(jnp.int32) + c * CHUNK
        upd = m > best_val
        return jnp.where(upd, m, best_val), jnp.where(upd, a, best_idx)

    init = (jnp.full((ROW_TILE,), -jnp.inf, jnp.float32),
            jnp.zeros((ROW_TILE,), jnp.int32))
    best_val, best_idx = lax.fori_loop(0, N_CHUNKS, body, init)
    idx_ref[0, 0, :] = best_idx

    part = (jnp.sum(normsq)
            - 2.0 * jnp.sum(best_val * norm[:, 0])
            + jnp.float32(ROW_TILE))

    @pl.when(i == 0)
    def _():
        loss_ref[0, 0] = 0.0

    loss_ref[0, 0] += part


_tc_call = pl.pallas_call(
    _tc_body,
    grid=(N_ROW_TILES,),
    in_specs=[
        pl.BlockSpec((ROW_TILE, DIM), lambda i: (i, 0)),
        pl.BlockSpec((NUM_EMB, DIM), lambda i: (0, 0)),
    ],
    out_specs=[
        pl.BlockSpec((1, 1, ROW_TILE), lambda i: (i, 0, 0)),
        pl.BlockSpec(memory_space=pltpu.SMEM),
    ],
    out_shape=[
        jax.ShapeDtypeStruct((N_ROW_TILES, 1, ROW_TILE), jnp.int32),
        jax.ShapeDtypeStruct((1, 1), jnp.float32),
    ],
)


_NC = 2
_NS = 16
_NW = _NC * _NS
_B_PER_W = ROWS // _NW

@functools.cache
def _make_sc_gather():
    mesh = plsc.VectorSubcoreMesh(core_axis_name="c", subcore_axis_name="s")

    @functools.partial(
        pl.kernel,
        mesh=mesh,
        compiler_params=pltpu.CompilerParams(use_tc_tiling_on_sc=False),
        out_type=jax.ShapeDtypeStruct((ROWS, DIM), jnp.float32),
        scratch_types=[
            pltpu.VMEM((_B_PER_W,), jnp.int32),
            pltpu.VMEM((_B_PER_W, DIM), jnp.float32),
            pltpu.SemaphoreType.DMA,
        ],
    )
    def _sc_gather(table_hbm, idx_hbm, out_hbm, idx_v, rows_v, sem):
        wid = lax.axis_index("s") * _NC + lax.axis_index("c")
        base = wid * _B_PER_W
        pltpu.sync_copy(idx_hbm.at[pl.ds(base, _B_PER_W)], idx_v)
        pltpu.async_copy(table_hbm.at[idx_v], rows_v, sem).wait()
        pltpu.sync_copy(rows_v, out_hbm.at[pl.ds(base, _B_PER_W)])

    return _sc_gather


def kernel(inputs, embeddings):
    input_shape = inputs.shape
    flat = inputs.reshape(ROWS, DIM)
    idx3, loss_acc = _tc_call(flat, embeddings)
    idx = idx3.reshape(ROWS)
    quantized = _make_sc_gather()(embeddings, idx)
    q = quantized.reshape(input_shape)
    loss = loss_acc[0, 0] * jnp.float32(1.0 / (ROWS * DIM))
    quantized_st = inputs + lax.stop_gradient(q - inputs)
    return (quantized_st, loss, idx.reshape(input_shape[:-1]))


# speedup vs baseline: 1.6434x; 1.5335x over previous
import functools

import jax
import jax.numpy as jnp
from jax import lax
from jax.experimental import pallas as pl
from jax.experimental.pallas import tpu as pltpu
from jax.experimental.pallas import tpu_sc as plsc

NUM_EMB = 8192
DIM = 64
ROWS = 16384
ROW_TILE = 512
N_ROW_TILES = ROWS // ROW_TILE
_DOT_PRECISION = lax.Precision.DEFAULT


def _tc_body(x_ref, emb_ref, idx_ref, loss_ref):
    i = pl.program_id(0)
    x = x_ref[...]
    normsq = jnp.sum(x * x, axis=1, keepdims=True)
    norm = jnp.sqrt(normsq)
    xn = x / jnp.maximum(norm, 1e-12)

    d = lax.dot_general(
        xn, emb_ref[...], (((1,), (1,)), ((), ())),
        preferred_element_type=jnp.float32,
        precision=_DOT_PRECISION,
    )
    best_val = jnp.max(d, axis=1)
    best_idx = jnp.argmax(d, axis=1).astype(jnp.int32)
    idx_ref[0, 0, :] = best_idx

    part = (jnp.sum(normsq)
            - 2.0 * jnp.sum(best_val * norm[:, 0])
            + jnp.float32(ROW_TILE))

    @pl.when(i == 0)
    def _():
        loss_ref[0, 0] = 0.0

    loss_ref[0, 0] += part


_tc_call = pl.pallas_call(
    _tc_body,
    grid=(N_ROW_TILES,),
    in_specs=[
        pl.BlockSpec((ROW_TILE, DIM), lambda i: (i, 0)),
        pl.BlockSpec((NUM_EMB, DIM), lambda i: (0, 0)),
    ],
    out_specs=[
        pl.BlockSpec((1, 1, ROW_TILE), lambda i: (i, 0, 0)),
        pl.BlockSpec(memory_space=pltpu.SMEM),
    ],
    out_shape=[
        jax.ShapeDtypeStruct((N_ROW_TILES, 1, ROW_TILE), jnp.int32),
        jax.ShapeDtypeStruct((1, 1), jnp.float32),
    ],
)


_NC = 2
_NS = 16
_NW = _NC * _NS
_B_PER_W = ROWS // _NW

@functools.cache
def _make_sc_gather():
    mesh = plsc.VectorSubcoreMesh(core_axis_name="c", subcore_axis_name="s")

    @functools.partial(
        pl.kernel,
        mesh=mesh,
        compiler_params=pltpu.CompilerParams(use_tc_tiling_on_sc=False),
        out_type=jax.ShapeDtypeStruct((ROWS, DIM), jnp.float32),
        scratch_types=[
            pltpu.VMEM((_B_PER_W,), jnp.int32),
            pltpu.VMEM((_B_PER_W, DIM), jnp.float32),
            pltpu.SemaphoreType.DMA,
        ],
    )
    def _sc_gather(table_hbm, idx_hbm, out_hbm, idx_v, rows_v, sem):
        wid = lax.axis_index("s") * _NC + lax.axis_index("c")
        base = wid * _B_PER_W
        pltpu.sync_copy(idx_hbm.at[pl.ds(base, _B_PER_W)], idx_v)
        pltpu.async_copy(table_hbm.at[idx_v], rows_v, sem).wait()
        pltpu.sync_copy(rows_v, out_hbm.at[pl.ds(base, _B_PER_W)])

    return _sc_gather


def kernel(inputs, embeddings):
    input_shape = inputs.shape
    flat = inputs.reshape(ROWS, DIM)
    idx3, loss_acc = _tc_call(flat, embeddings)
    idx = idx3.reshape(ROWS)
    quantized = _make_sc_gather()(embeddings, idx)
    q = quantized.reshape(input_shape)
    loss = loss_acc[0, 0] * jnp.float32(1.0 / (ROWS * DIM))
    quantized_st = inputs + lax.stop_gradient(q - inputs)
    return (quantized_st, loss, idx.reshape(input_shape[:-1]))


# speedup vs baseline: 1.7672x; 1.0753x over previous
import functools

import jax
import jax.numpy as jnp
from jax import lax
from jax.experimental import pallas as pl
from jax.experimental.pallas import tpu as pltpu
from jax.experimental.pallas import tpu_sc as plsc

NUM_EMB = 8192
DIM = 64
ROWS = 16384
ROW_TILE = 512
N_ROW_TILES = ROWS // ROW_TILE
_DOT_PRECISION = lax.Precision.DEFAULT


def _tc_body(x_ref, emb_ref, idx_ref):
    x = x_ref[...]
    normsq = jnp.sum(x * x, axis=1, keepdims=True)
    norm = jnp.sqrt(normsq)
    xn = x / jnp.maximum(norm, 1e-12)

    d = lax.dot_general(
        xn, emb_ref[...], (((1,), (1,)), ((), ())),
        preferred_element_type=jnp.float32,
        precision=_DOT_PRECISION,
    )
    best_idx = jnp.argmax(d, axis=1).astype(jnp.int32)
    idx_ref[0, 0, :] = best_idx


_tc_call = pl.pallas_call(
    _tc_body,
    grid=(N_ROW_TILES,),
    in_specs=[
        pl.BlockSpec((ROW_TILE, DIM), lambda i: (i, 0)),
        pl.BlockSpec((NUM_EMB, DIM), lambda i: (0, 0)),
    ],
    out_specs=pl.BlockSpec((1, 1, ROW_TILE), lambda i: (i, 0, 0)),
    out_shape=jax.ShapeDtypeStruct((N_ROW_TILES, 1, ROW_TILE), jnp.int32),
)


_NC = 2
_NS = 16
_NW = _NC * _NS
_B_PER_W = ROWS // _NW
_LANES = 16
_GROUPS = DIM // _LANES


@functools.cache
def _make_sc_gather():
    mesh = plsc.VectorSubcoreMesh(core_axis_name="c", subcore_axis_name="s")

    @functools.partial(
        pl.kernel,
        mesh=mesh,
        compiler_params=pltpu.CompilerParams(use_tc_tiling_on_sc=False),
        out_type=(
            jax.ShapeDtypeStruct((ROWS, DIM), jnp.float32),
            jax.ShapeDtypeStruct((_NW, _LANES), jnp.float32),
        ),
        scratch_types=[
            pltpu.VMEM((_B_PER_W,), jnp.int32),
            pltpu.VMEM((_B_PER_W, DIM), jnp.float32),
            pltpu.VMEM((_B_PER_W, DIM), jnp.float32),
            pltpu.VMEM((_LANES,), jnp.float32),
            pltpu.SemaphoreType.DMA,
        ],
    )
    def _sc_gather(table_hbm, idx_hbm, x_hbm, out_hbm, loss_hbm,
                   idx_v, rows_v, x_v, acc_v, sem):
        wid = lax.axis_index("s") * _NC + lax.axis_index("c")
        base = wid * _B_PER_W
        pltpu.sync_copy(idx_hbm.at[pl.ds(base, _B_PER_W)], idx_v)
        copy = pltpu.async_copy(table_hbm.at[idx_v], rows_v, sem)
        pltpu.sync_copy(x_hbm.at[pl.ds(base, _B_PER_W)], x_v)
        copy.wait()
        pltpu.sync_copy(rows_v, out_hbm.at[pl.ds(base, _B_PER_W)])

        def body(r, acc):
            for g in range(_GROUPS):
                q = rows_v[r, pl.ds(g * _LANES, _LANES)]
                xv = x_v[r, pl.ds(g * _LANES, _LANES)]
                diff = q - xv
                acc = acc + diff * diff
            return acc

        acc = lax.fori_loop(0, _B_PER_W, body,
                            jnp.zeros((_LANES,), jnp.float32))
        acc_v[...] = acc
        pltpu.sync_copy(acc_v, loss_hbm.at[wid])

    return _sc_gather


def kernel(inputs, embeddings):
    input_shape = inputs.shape
    flat = inputs.reshape(ROWS, DIM)
    idx3 = _tc_call(flat, embeddings)
    idx = idx3.reshape(ROWS)
    quantized, loss_part = _make_sc_gather()(embeddings, idx, flat)
    q = quantized.reshape(input_shape)
    loss = jnp.sum(loss_part) * jnp.float32(1.0 / (ROWS * DIM))
    quantized_st = inputs + lax.stop_gradient(q - inputs)
    return (quantized_st, loss, idx.reshape(input_shape[:-1]))


# speedup vs baseline: 1.7900x; 1.0129x over previous
import functools

import jax
import jax.numpy as jnp
from jax import lax
from jax.experimental import pallas as pl
from jax.experimental.pallas import tpu as pltpu
from jax.experimental.pallas import tpu_sc as plsc

NUM_EMB = 8192
DIM = 64
ROWS = 16384
ROW_TILE = 1024
N_ROW_TILES = ROWS // ROW_TILE
_DOT_PRECISION = lax.Precision.DEFAULT


def _tc_body(x_ref, emb_ref, idx_ref):
    x = x_ref[...]
    normsq = jnp.sum(x * x, axis=1, keepdims=True)
    norm = jnp.sqrt(normsq)
    xn = x / jnp.maximum(norm, 1e-12)

    d = lax.dot_general(
        xn, emb_ref[...], (((1,), (1,)), ((), ())),
        preferred_element_type=jnp.float32,
        precision=_DOT_PRECISION,
    )
    best_idx = jnp.argmax(d, axis=1).astype(jnp.int32)
    idx_ref[0, 0, :] = best_idx


_tc_call = pl.pallas_call(
    _tc_body,
    grid=(N_ROW_TILES,),
    in_specs=[
        pl.BlockSpec((ROW_TILE, DIM), lambda i: (i, 0)),
        pl.BlockSpec((NUM_EMB, DIM), lambda i: (0, 0)),
    ],
    out_specs=pl.BlockSpec((1, 1, ROW_TILE), lambda i: (i, 0, 0)),
    out_shape=jax.ShapeDtypeStruct((N_ROW_TILES, 1, ROW_TILE), jnp.int32),
)


_NC = 2
_NS = 16
_NW = _NC * _NS
_B_PER_W = ROWS // _NW
_LANES = 16
_GROUPS = DIM // _LANES


@functools.cache
def _make_sc_gather():
    mesh = plsc.VectorSubcoreMesh(core_axis_name="c", subcore_axis_name="s")

    @functools.partial(
        pl.kernel,
        mesh=mesh,
        compiler_params=pltpu.CompilerParams(use_tc_tiling_on_sc=False),
        out_type=(
            jax.ShapeDtypeStruct((ROWS, DIM), jnp.float32),
            jax.ShapeDtypeStruct((_NW, _LANES), jnp.float32),
        ),
        scratch_types=[
            pltpu.VMEM((_B_PER_W,), jnp.int32),
            pltpu.VMEM((_B_PER_W, DIM), jnp.float32),
            pltpu.VMEM((_B_PER_W, DIM), jnp.float32),
            pltpu.VMEM((_LANES,), jnp.float32),
            pltpu.SemaphoreType.DMA,
        ],
    )
    def _sc_gather(table_hbm, idx_hbm, x_hbm, out_hbm, loss_hbm,
                   idx_v, rows_v, x_v, acc_v, sem):
        wid = lax.axis_index("s") * _NC + lax.axis_index("c")
        base = wid * _B_PER_W
        pltpu.sync_copy(idx_hbm.at[pl.ds(base, _B_PER_W)], idx_v)
        copy = pltpu.async_copy(table_hbm.at[idx_v], rows_v, sem)
        pltpu.sync_copy(x_hbm.at[pl.ds(base, _B_PER_W)], x_v)
        copy.wait()
        pltpu.sync_copy(rows_v, out_hbm.at[pl.ds(base, _B_PER_W)])

        def body(r, acc):
            for g in range(_GROUPS):
                q = rows_v[r, pl.ds(g * _LANES, _LANES)]
                xv = x_v[r, pl.ds(g * _LANES, _LANES)]
                diff = q - xv
                acc = acc + diff * diff
            return acc

        acc = lax.fori_loop(0, _B_PER_W, body,
                            jnp.zeros((_LANES,), jnp.float32))
        acc_v[...] = acc
        pltpu.sync_copy(acc_v, loss_hbm.at[wid])

    return _sc_gather


def kernel(inputs, embeddings):
    input_shape = inputs.shape
    flat = inputs.reshape(ROWS, DIM)
    idx3 = _tc_call(flat, embeddings)
    idx = idx3.reshape(ROWS)
    quantized, loss_part = _make_sc_gather()(embeddings, idx, flat)
    q = quantized.reshape(input_shape)
    loss = jnp.sum(loss_part) * jnp.float32(1.0 / (ROWS * DIM))
    quantized_st = inputs + lax.stop_gradient(q - inputs)
    return (quantized_st, loss, idx.reshape(input_shape[:-1]))


# speedup vs baseline: 1.8500x; 1.0335x over previous
import functools

import jax
import jax.numpy as jnp
from jax import lax
from jax.experimental import pallas as pl
from jax.experimental.pallas import tpu as pltpu
from jax.experimental.pallas import tpu_sc as plsc

NUM_EMB = 8192
DIM = 64
ROWS = 16384
ROW_TILE = 1024
N_ROW_TILES = ROWS // ROW_TILE
_DOT_PRECISION = lax.Precision.DEFAULT


def _tc_body(x_ref, emb_ref, idx_ref):
    x = x_ref[...]
    normsq = jnp.sum(x * x, axis=1, keepdims=True)
    norm = jnp.sqrt(normsq)
    xn = x / jnp.maximum(norm, 1e-12)

    d = lax.dot_general(
        xn, emb_ref[...], (((1,), (1,)), ((), ())),
        preferred_element_type=jnp.float32,
        precision=_DOT_PRECISION,
    )
    best_idx = jnp.argmax(d, axis=1).astype(jnp.int32)
    idx_ref[0, 0, :] = best_idx


_tc_call = pl.pallas_call(
    _tc_body,
    grid=(N_ROW_TILES,),
    in_specs=[
        pl.BlockSpec((ROW_TILE, DIM), lambda i: (i, 0)),
        pl.BlockSpec((NUM_EMB, DIM), lambda i: (0, 0)),
    ],
    out_specs=pl.BlockSpec((1, 1, ROW_TILE), lambda i: (i, 0, 0)),
    out_shape=jax.ShapeDtypeStruct((N_ROW_TILES, 1, ROW_TILE), jnp.int32),
)


_NC = 2
_NS = 16
_NW = _NC * _NS
_B_PER_W = ROWS // _NW
_LANES = 16
_GROUPS = DIM // _LANES


@functools.cache
def _make_sc_gather():
    mesh = plsc.VectorSubcoreMesh(core_axis_name="c", subcore_axis_name="s")

    @functools.partial(
        pl.kernel,
        mesh=mesh,
        compiler_params=pltpu.CompilerParams(use_tc_tiling_on_sc=False),
        out_type=(
            jax.ShapeDtypeStruct((ROWS, DIM), jnp.float32),
            jax.ShapeDtypeStruct((_NW, _LANES), jnp.float32),
        ),
        scratch_types=[
            pltpu.VMEM((_B_PER_W,), jnp.int32),
            pltpu.VMEM((_B_PER_W, DIM), jnp.float32),
            pltpu.VMEM((_B_PER_W, DIM), jnp.float32),
            pltpu.VMEM((_LANES,), jnp.float32),
            pltpu.SemaphoreType.DMA,
        ],
    )
    def _sc_gather(table_hbm, idx_hbm, x_hbm, out_hbm, loss_hbm,
                   idx_v, rows_v, x_v, acc_v, sem):
        wid = lax.axis_index("s") * _NC + lax.axis_index("c")
        base = wid * _B_PER_W
        pltpu.sync_copy(idx_hbm.at[pl.ds(base, _B_PER_W)], idx_v)
        copy = pltpu.async_copy(table_hbm.at[idx_v], rows_v, sem)
        pltpu.sync_copy(x_hbm.at[pl.ds(base, _B_PER_W)], x_v)
        copy.wait()

        def body(r, acc):
            for g in range(_GROUPS):
                q = rows_v[r, pl.ds(g * _LANES, _LANES)]
                xv = x_v[r, pl.ds(g * _LANES, _LANES)]
                diff = q - xv
                acc = acc + diff * diff
                rows_v[r, pl.ds(g * _LANES, _LANES)] = xv + diff
            return acc

        acc = lax.fori_loop(0, _B_PER_W, body,
                            jnp.zeros((_LANES,), jnp.float32))
        acc_v[...] = acc
        pltpu.sync_copy(rows_v, out_hbm.at[pl.ds(base, _B_PER_W)])
        pltpu.sync_copy(acc_v, loss_hbm.at[wid])

    return _sc_gather


def kernel(inputs, embeddings):
    input_shape = inputs.shape
    flat = inputs.reshape(ROWS, DIM)
    idx3 = _tc_call(flat, embeddings)
    idx = idx3.reshape(ROWS)
    quantized_st, loss_part = _make_sc_gather()(embeddings, idx, flat)
    loss = jnp.sum(loss_part) * jnp.float32(1.0 / (ROWS * DIM))
    return (quantized_st.reshape(input_shape), loss,
            idx.reshape(input_shape[:-1]))
